# baseline (speedup 1.0000x reference)
import functools

import jax
import jax.numpy as jnp
from jax import lax
from jax.experimental import pallas as pl
from jax.experimental.pallas import tpu as pltpu
from jax.experimental.pallas import tpu_sc as plsc

B, S, H = 4, 2048, 1024
N = B * S
NW = 32
TPW = N // NW
C = 32
NCHUNK = TPW // C
LANES = 16
GPT = H // LANES

_mesh = plsc.VectorSubcoreMesh(core_axis_name="c", subcore_axis_name="s")


@functools.partial(
    pl.kernel,
    mesh=_mesh,
    out_type=jax.ShapeDtypeStruct((N, H), jnp.float32),
    scratch_types=[
        pltpu.VMEM((C,), jnp.int32),
        pltpu.VMEM((C,), jnp.int32),
        pltpu.VMEM((C, H), jnp.float32),
        pltpu.VMEM((C, H), jnp.float32),
        pltpu.VMEM((C, H), jnp.float32),
        pltpu.SemaphoreType.DMA,
    ],
)
def _sc_embed(ids_hbm, tt_hbm, word_hbm, pos_hbm, type_hbm, out_hbm,
              idx_v, ttx_v, wbuf, pbuf, tbuf, sem):
    wid = lax.axis_index("s") * 2 + lax.axis_index("c")
    tok0 = wid * TPW
    s0 = tok0 % S

    for k in range(NCHUNK):
        base = tok0 + k * C
        pltpu.sync_copy(ids_hbm.at[pl.ds(base, C)], idx_v)
        pltpu.sync_copy(tt_hbm.at[pl.ds(base, C)], ttx_v)
        cp_w = pltpu.async_copy(word_hbm.at[idx_v], wbuf, sem)
        cp_t = pltpu.async_copy(type_hbm.at[ttx_v], tbuf, sem)
        pltpu.sync_copy(pos_hbm.at[pl.ds(s0 + k * C, C)], pbuf)
        cp_w.wait()
        cp_t.wait()

        def body(t, carry):
            for j in range(GPT):
                sl = pl.ds(j * LANES, LANES)
                wbuf[t, sl] = wbuf[t, sl] + pbuf[t, sl] + tbuf[t, sl]
            return carry

        lax.fori_loop(0, C, body, 0)
        pltpu.sync_copy(wbuf, out_hbm.at[pl.ds(base, C)])


def kernel(input_ids, token_type_ids, word_embeddings, position_embeddings,
           token_type_embeddings):
    ids = input_ids.reshape(N).astype(jnp.int32)
    tts = token_type_ids.reshape(N).astype(jnp.int32)
    out = _sc_embed(ids, tts, word_embeddings, position_embeddings,
                    token_type_embeddings)
    return out.reshape(B, S, H)


# speedup vs baseline: 2.3705x; 2.3705x over previous
import functools

import jax
import jax.numpy as jnp
from jax import lax
from jax.experimental import pallas as pl
from jax.experimental.pallas import tpu as pltpu
from jax.experimental.pallas import tpu_sc as plsc

B, S, H = 4, 2048, 1024
T = 2
N = B * S
NW = 32
TPW = N // NW
C = 16
NCHUNK = TPW // C
LANES = 16
GPT = H // LANES
SBLK = 256

_mesh = plsc.VectorSubcoreMesh(core_axis_name="c", subcore_axis_name="s")


def _combo_body(pos_ref, type_ref, out_ref):
    t = pl.program_id(0)
    out_ref[...] = pos_ref[...] + type_ref[pl.ds(t, 1), :]


def _build_combo(pos, typ):
    grid = (T, S // SBLK)
    return pl.pallas_call(
        _combo_body,
        grid=grid,
        in_specs=[
            pl.BlockSpec((SBLK, H), lambda t, j: (j, 0)),
            pl.BlockSpec((T, H), lambda t, j: (0, 0)),
        ],
        out_specs=pl.BlockSpec((SBLK, H), lambda t, j: (t * (S // SBLK) + j, 0)),
        out_shape=jax.ShapeDtypeStruct((T * S, H), jnp.float32),
    )(pos[:S], typ)


@functools.partial(
    pl.kernel,
    mesh=_mesh,
    out_type=jax.ShapeDtypeStruct((N, H), jnp.float32),
    scratch_types=[
        pltpu.VMEM((TPW,), jnp.int32),
        pltpu.VMEM((TPW,), jnp.int32),
        pltpu.VMEM((2, C, H), jnp.float32),
        pltpu.VMEM((4, C, H), jnp.float32),
        pltpu.SemaphoreType.DMA,
        pltpu.SemaphoreType.DMA,
        pltpu.SemaphoreType.DMA,
        pltpu.SemaphoreType.DMA,
        pltpu.SemaphoreType.DMA,
        pltpu.SemaphoreType.DMA,
        pltpu.SemaphoreType.DMA,
        pltpu.SemaphoreType.DMA,
        pltpu.SemaphoreType.DMA,
        pltpu.SemaphoreType.DMA,
    ],
)
def _sc_embed(ids_hbm, tt_hbm, word_hbm, combo_hbm, out_hbm,
              idx_v, cidx_v, wbuf, obuf,
              w0, w1, p0, p1, p2, p3, o0, o1, o2, o3):
    wid = lax.axis_index("s") * 2 + lax.axis_index("c")
    tok0 = wid * TPW
    s0 = tok0 % S
    wsem = (w0, w1)
    psem = (p0, p1, p2, p3)
    osem = (o0, o1, o2, o3)

    pltpu.sync_copy(ids_hbm.at[pl.ds(tok0, TPW)], idx_v)
    pltpu.sync_copy(tt_hbm.at[pl.ds(tok0, TPW)], cidx_v)

    lane = lax.iota(jnp.int32, LANES)
    for g in range(TPW // LANES):
        sl = pl.ds(g * LANES, LANES)
        cidx_v[sl] = cidx_v[sl] * S + (s0 + g * LANES) + lane

    def start_word(k):
        idx = idx_v.at[pl.ds(k * C, C)]
        return pltpu.async_copy(word_hbm.at[idx], wbuf.at[k % 2], wsem[k % 2])

    def start_combo(k):
        cidx = cidx_v.at[pl.ds(k * C, C)]
        return pltpu.async_copy(combo_hbm.at[cidx], obuf.at[k % 4],
                                psem[k % 4])

    def compute_chunk(k):
        ws, os_ = k % 2, k % 4

        def trow(t, carry):
            def grp(g, carry2):
                sl = pl.ds(g * LANES, LANES)
                plsc.addupdate(obuf.at[os_, t, sl], wbuf[ws, t, sl])
                return carry2

            return lax.fori_loop(0, GPT, grp, carry, unroll=16)

        lax.fori_loop(0, C, trow, 0)

    wcp = {0: start_word(0), 1: start_word(1)}
    pcp = {0: start_combo(0), 1: start_combo(1), 2: start_combo(2)}
    ocp = {}
    for k in range(NCHUNK):
        wcp.pop(k).wait()
        pcp.pop(k).wait()
        compute_chunk(k)
        ocp[k] = pltpu.async_copy(obuf.at[k % 4],
                                  out_hbm.at[pl.ds(tok0 + k * C, C)],
                                  osem[k % 4])
        if k + 2 < NCHUNK:
            wcp[k + 2] = start_word(k + 2)
        if k + 3 < NCHUNK:
            if k - 1 in ocp:
                ocp.pop(k - 1).wait()
            pcp[k + 3] = start_combo(k + 3)
    for d in ocp.values():
        d.wait()


def kernel(input_ids, token_type_ids, word_embeddings, position_embeddings,
           token_type_embeddings):
    ids = input_ids.reshape(N).astype(jnp.int32)
    tts = token_type_ids.reshape(N).astype(jnp.int32)
    combo = _build_combo(position_embeddings, token_type_embeddings)
    out = _sc_embed(ids, tts, word_embeddings, combo)
    return out.reshape(B, S, H)


# speedup vs baseline: 3.2587x; 1.3747x over previous
import functools

import jax
import jax.numpy as jnp
from jax import lax
from jax.experimental import pallas as pl
from jax.experimental.pallas import tpu as pltpu
from jax.experimental.pallas import tpu_sc as plsc

B, S, H = 4, 2048, 1024
T = 2
N = B * S
NW = 32
TPW = N // NW
C = 32
NCHUNK = TPW // C
NBLK = 512

_mesh = plsc.VectorSubcoreMesh(core_axis_name="c", subcore_axis_name="s")


@functools.partial(
    pl.kernel,
    mesh=_mesh,
    out_type=jax.ShapeDtypeStruct((N, H), jnp.float32),
    scratch_types=[
        pltpu.VMEM((TPW,), jnp.int32),
        pltpu.VMEM((2, C, H), jnp.float32),
        pltpu.SemaphoreType.DMA,
        pltpu.SemaphoreType.DMA,
        pltpu.SemaphoreType.DMA,
        pltpu.SemaphoreType.DMA,
    ],
)
def _sc_gather(ids_hbm, word_hbm, out_hbm, idx_v, wbuf, g0, g1, o0, o1):
    wid = lax.axis_index("s") * 2 + lax.axis_index("c")
    tok0 = wid * TPW
    gsem = (g0, g1)
    osem = (o0, o1)

    pltpu.sync_copy(ids_hbm.at[pl.ds(tok0, TPW)], idx_v)

    def start_gather(k):
        idx = idx_v.at[pl.ds(k * C, C)]
        return pltpu.async_copy(word_hbm.at[idx], wbuf.at[k % 2], gsem[k % 2])

    gcp = {0: start_gather(0), 1: start_gather(1)}
    ocp = {}
    for k in range(NCHUNK):
        gcp.pop(k).wait()
        ocp[k] = pltpu.async_copy(wbuf.at[k % 2],
                                  out_hbm.at[pl.ds(tok0 + k * C, C)],
                                  osem[k % 2])
        if k + 2 < NCHUNK:
            ocp.pop(k).wait()
            gcp[k + 2] = start_gather(k + 2)
    for d in ocp.values():
        d.wait()


def _add_body(w_ref, pos_ref, typ_ref, tt_ref, out_ref):
    mask = tt_ref[0, :, :] == 0
    typed = jnp.where(mask, typ_ref[0:1, :], typ_ref[1:2, :])
    out_ref[...] = w_ref[...] + pos_ref[...] + typed


def _tc_add(w, pos, typ, tts):
    grid = (N // NBLK,)
    tt3 = tts.reshape(N // NBLK, NBLK, 1)
    return pl.pallas_call(
        _add_body,
        grid=grid,
        in_specs=[
            pl.BlockSpec((NBLK, H), lambda j: (j, 0)),
            pl.BlockSpec((NBLK, H), lambda j: (j % (S // NBLK), 0)),
            pl.BlockSpec((T, H), lambda j: (0, 0)),
            pl.BlockSpec((1, NBLK, 1), lambda j: (j, 0, 0)),
        ],
        out_specs=pl.BlockSpec((NBLK, H), lambda j: (j, 0)),
        out_shape=jax.ShapeDtypeStruct((N, H), jnp.float32),
    )(w, pos[:S], typ, tt3)


def kernel(input_ids, token_type_ids, word_embeddings, position_embeddings,
           token_type_embeddings):
    ids = input_ids.reshape(N).astype(jnp.int32)
    tts = token_type_ids.reshape(N).astype(jnp.int32)
    w = _sc_gather(ids, word_embeddings)
    out = _tc_add(w, position_embeddings, token_type_embeddings, tts)
    return out.reshape(B, S, H)


# speedup vs baseline: 3.4601x; 1.0618x over previous
import functools

import jax
import jax.numpy as jnp
from jax import lax
from jax.experimental import pallas as pl
from jax.experimental.pallas import tpu as pltpu
from jax.experimental.pallas import tpu_sc as plsc

B, S, H = 4, 2048, 1024
T = 2
N = B * S
NW = 32
TPW = N // NW
C = 32
NCHUNK = TPW // C
NBLK = 512

_mesh = plsc.VectorSubcoreMesh(core_axis_name="c", subcore_axis_name="s")


@functools.partial(
    pl.kernel,
    mesh=_mesh,
    out_type=jax.ShapeDtypeStruct((N, H), jnp.float32),
    scratch_types=[
        pltpu.VMEM((TPW,), jnp.int32),
        pltpu.VMEM((2, C, H), jnp.float32),
        pltpu.SemaphoreType.DMA,
        pltpu.SemaphoreType.DMA,
        pltpu.SemaphoreType.DMA,
        pltpu.SemaphoreType.DMA,
    ],
)
def _sc_gather(ids_hbm, word_hbm, out_hbm, idx_v, wbuf, g0, g1, o0, o1):
    wid = lax.axis_index("s") * 2 + lax.axis_index("c")
    tok0 = wid * TPW
    gsem = (g0, g1)
    osem = (o0, o1)

    pltpu.sync_copy(ids_hbm.at[pl.ds(tok0, TPW)], idx_v)

    def start_gather(k):
        idx = idx_v.at[pl.ds(k * C, C)]
        return pltpu.async_copy(word_hbm.at[idx], wbuf.at[k % 2], gsem[k % 2])

    gcp = {0: start_gather(0), 1: start_gather(1)}
    ocp = {}
    for k in range(NCHUNK):
        gcp.pop(k).wait()
        ocp[k] = pltpu.async_copy(wbuf.at[k % 2],
                                  out_hbm.at[pl.ds(tok0 + k * C, C)],
                                  osem[k % 2])
        if k + 2 < NCHUNK:
            ocp.pop(k).wait()
            gcp[k + 2] = start_gather(k + 2)
    for d in ocp.values():
        d.wait()


def _add_body(w_ref, pos_ref, typ_ref, tt_ref, out_ref):
    mask = tt_ref[0, :, :] == 0
    typed = jnp.where(mask, typ_ref[0:1, :], typ_ref[1:2, :])
    out_ref[...] = w_ref[...] + pos_ref[...] + typed


def _tc_add(w, pos, typ, tts):
    grid = (S // NBLK, B)
    tt3 = tts.reshape(N // NBLK, NBLK, 1)
    nj = S // NBLK
    return pl.pallas_call(
        _add_body,
        grid=grid,
        in_specs=[
            pl.BlockSpec((NBLK, H), lambda j, b: (b * nj + j, 0)),
            pl.BlockSpec((NBLK, H), lambda j, b: (j, 0)),
            pl.BlockSpec((T, H), lambda j, b: (0, 0)),
            pl.BlockSpec((1, NBLK, 1), lambda j, b: (b * nj + j, 0, 0)),
        ],
        out_specs=pl.BlockSpec((NBLK, H), lambda j, b: (b * nj + j, 0)),
        out_shape=jax.ShapeDtypeStruct((N, H), jnp.float32),
    )(w, pos[:S], typ, tt3)


def kernel(input_ids, token_type_ids, word_embeddings, position_embeddings,
           token_type_embeddings):
    ids = input_ids.reshape(N).astype(jnp.int32)
    tts = token_type_ids.reshape(N).astype(jnp.int32)
    w = _sc_gather(ids, word_embeddings)
    out = _tc_add(w, position_embeddings, token_type_embeddings, tts)
    return out.reshape(B, S, H)
